# baseline (speedup 1.0000x reference)
import functools
import math

import jax
import jax.numpy as jnp
from jax import lax
from jax.experimental import pallas as pl
from jax.experimental.pallas import tpu as pltpu
from jax.experimental.pallas import tpu_sc as plsc

_BATCH = 512
_SEQ = 1024
_EMB = 128
_HALF = _EMB // 2

_NC = 2
_NS = 16
_NW = _NC * _NS
_RPW = _BATCH // _NW
_LANES = 16


def _chase_body(sol_hbm, vt_hbm, sol_v, vt_v):
    wid = lax.axis_index("s") * _NC + lax.axis_index("c")
    base = wid * _RPW
    pltpu.sync_copy(sol_hbm.at[pl.ds(base, _RPW)], sol_v)

    lanes = lax.iota(jnp.int32, _LANES)
    zeros = jnp.zeros_like(lanes)

    def _zero(j, carry):
        for r in range(_RPW):
            vt_v[r, pl.ds(j * _LANES, _LANES)] = zeros
        return carry

    lax.fori_loop(0, _SEQ // _LANES, _zero, 0, unroll=False)

    def _step(i, pre):
        cur = plsc.load_gather(sol_v, [lanes, pre])
        plsc.store_scatter(vt_v, [lanes, cur], jnp.broadcast_to(i + 1, (_LANES,)))
        return cur

    lax.fori_loop(0, _SEQ + 2, _step, zeros, unroll=False)
    pltpu.sync_copy(vt_v, vt_hbm.at[pl.ds(base, _RPW)])


def _make_chase():
    mesh = plsc.VectorSubcoreMesh(
        core_axis_name="c", subcore_axis_name="s", num_cores=_NC, num_subcores=_NS
    )
    return pl.kernel(
        _chase_body,
        out_type=jax.ShapeDtypeStruct((_BATCH, _SEQ), jnp.int32),
        mesh=mesh,
        scratch_types=[
            pltpu.VMEM((_RPW, _SEQ), jnp.int32),
            pltpu.VMEM((_RPW, _SEQ), jnp.int32),
        ],
        compiler_params=pltpu.CompilerParams(
            use_tc_tiling_on_sc=False, needs_layout_passes=False
        ),
    )


_BB = 8
_BS = 512


def _dense_body(x0_ref, x1_ref, vt_ref, c_ref, f_ref, emb_ref, re_ref, im_ref):
    x0 = x0_ref[...]
    x1 = x1_ref[...]
    w0 = c_ref[0, :]
    w1 = c_ref[1, :]
    emb_ref[...] = (
        x0[:, :, None] * w0[None, None, :] + x1[:, :, None] * w1[None, None, :]
    )
    tr = c_ref[2, 0]
    t2 = jnp.mod(vt_ref[...].astype(jnp.float32), tr)
    f3 = f_ref[...][None, :, :]
    phase = jnp.broadcast_to(t2[:, None, :], (_BB, _HALF, _BS)) * f3
    re_ref[...] = jnp.cos(phase)
    im_ref[...] = jnp.cos(phase - (0.5 * math.pi))


def _dense(x0, x1, vt, consts, fmat):
    grid = (_BATCH // _BB, _SEQ // _BS)
    return pl.pallas_call(
        _dense_body,
        grid=grid,
        in_specs=[
            pl.BlockSpec((_BB, _BS), lambda i, j: (i, j)),
            pl.BlockSpec((_BB, _BS), lambda i, j: (i, j)),
            pl.BlockSpec((_BB, _BS), lambda i, j: (i, j)),
            pl.BlockSpec((8, _EMB), lambda i, j: (0, 0)),
            pl.BlockSpec((_HALF, _BS), lambda i, j: (0, j)),
        ],
        out_specs=[
            pl.BlockSpec((_BB, _BS, _EMB), lambda i, j: (i, j, 0)),
            pl.BlockSpec((_BB, _HALF, _BS), lambda i, j: (i, 0, j)),
            pl.BlockSpec((_BB, _HALF, _BS), lambda i, j: (i, 0, j)),
        ],
        out_shape=[
            jax.ShapeDtypeStruct((_BATCH, _SEQ, _EMB), jnp.float32),
            jax.ShapeDtypeStruct((_BATCH, _HALF, _SEQ), jnp.float32),
            jax.ShapeDtypeStruct((_BATCH, _HALF, _SEQ), jnp.float32),
        ],
        compiler_params=pltpu.CompilerParams(
            dimension_semantics=("parallel", "parallel"),
        ),
    )(x0, x1, vt, consts, fmat)


def kernel(x, solutions, step_info, W):
    dim = W.shape[0]
    visited_time = _make_chase()(solutions)

    x0 = x[:, :, 0]
    x1 = x[:, :, 1]

    freqs = 1.0 / (
        10000.0
        ** (jnp.arange(0, dim, 2, dtype=jnp.int32)[: dim // 2].astype(jnp.float32) / dim)
    )
    traced = (_SEQ - step_info[0] + 2 * step_info[1]).astype(jnp.float32)
    trv = jnp.broadcast_to(traced, (_EMB,))
    pad = jnp.zeros((5, _EMB), jnp.float32)
    consts = jnp.concatenate(
        [W[:, 0][None, :], W[:, 1][None, :], trv[None, :], pad], axis=0
    )
    fmat = jnp.broadcast_to(freqs[:, None], (_HALF, _SEQ))

    x_embedding, re3, im3 = _dense(x0, x1, visited_time, consts, fmat)
    freqs_cis = lax.complex(re3.transpose(0, 2, 1), im3.transpose(0, 2, 1))
    return (x_embedding, freqs_cis, visited_time)


# speedup vs baseline: 1.1754x; 1.1754x over previous
import functools
import math

import jax
import jax.numpy as jnp
from jax import lax
from jax.experimental import pallas as pl
from jax.experimental.pallas import tpu as pltpu
from jax.experimental.pallas import tpu_sc as plsc

_BATCH = 512
_SEQ = 1024
_EMB = 128
_HALF = _EMB // 2

_NC = 2
_NS = 16
_NW = _NC * _NS
_RPW = _BATCH // _NW
_LANES = 16


def _chase_body(sol_hbm, vt_hbm, sol_v, vt_v):
    wid = lax.axis_index("s") * _NC + lax.axis_index("c")
    base = wid * _RPW
    pltpu.sync_copy(sol_hbm.at[pl.ds(base, _RPW)], sol_v)

    lanes = lax.iota(jnp.int32, _LANES)
    zeros = jnp.zeros_like(lanes)

    def _zero(j, carry):
        for r in range(_RPW):
            vt_v[r, pl.ds(j * _LANES, _LANES)] = zeros
        return carry

    lax.fori_loop(0, _SEQ // _LANES, _zero, 0, unroll=False)

    def _step(i, pre):
        cur = plsc.load_gather(sol_v, [lanes, pre])
        plsc.store_scatter(vt_v, [lanes, cur], jnp.broadcast_to(i + 1, (_LANES,)))
        return cur

    lax.fori_loop(0, _SEQ + 2, _step, zeros, unroll=False)
    pltpu.sync_copy(vt_v, vt_hbm.at[pl.ds(base, _RPW)])


def _make_chase():
    mesh = plsc.VectorSubcoreMesh(
        core_axis_name="c", subcore_axis_name="s", num_cores=_NC, num_subcores=_NS
    )
    return pl.kernel(
        _chase_body,
        out_type=jax.ShapeDtypeStruct((_BATCH, _SEQ), jnp.int32),
        mesh=mesh,
        scratch_types=[
            pltpu.VMEM((_RPW, _SEQ), jnp.int32),
            pltpu.VMEM((_RPW, _SEQ), jnp.int32),
        ],
        compiler_params=pltpu.CompilerParams(
            use_tc_tiling_on_sc=False, needs_layout_passes=False
        ),
    )


_BB = 8
_BS = 512

_PIO2_HI = 1.5707855224609375
_PIO2_REST = 1.0804333896827965e-05
_2OPI = 0.6366197723675814
_S1, _S2, _S3 = -1.6666654611e-1, 8.3321608736e-3, -1.9515295891e-4
_C2, _C3, _C4 = 4.166664568298827e-2, -1.388731625493765e-3, 2.443315711809948e-5


def _sincos(phase):
    q = jnp.floor(phase * _2OPI + 0.5)
    r = (phase - q * _PIO2_HI) - q * _PIO2_REST
    r2 = r * r
    s = r + r * r2 * (_S1 + r2 * (_S2 + r2 * _S3))
    c = 1.0 + r2 * (-0.5 + r2 * (_C2 + r2 * (_C3 + r2 * _C4)))
    k = q.astype(jnp.int32) & 3
    k1 = k == 1
    k2 = k == 2
    k3 = k == 3
    cosv = jnp.where(k1, -s, jnp.where(k2, -c, jnp.where(k3, s, c)))
    sinv = jnp.where(k1, c, jnp.where(k2, -s, jnp.where(k3, -c, s)))
    return cosv, sinv


def _dense_body(x0_ref, x1_ref, vt_ref, c_ref, f_ref, emb_ref, re_ref, im_ref):
    x0 = x0_ref[...]
    x1 = x1_ref[...]
    w0 = c_ref[0, :]
    w1 = c_ref[1, :]
    emb_ref[...] = (
        x0[:, :, None] * w0[None, None, :] + x1[:, :, None] * w1[None, None, :]
    )
    tr = c_ref[2, 0]
    t2 = jnp.mod(vt_ref[...].astype(jnp.float32), tr)
    f3 = f_ref[...][None, :, :]
    phase = jnp.broadcast_to(t2[:, None, :], (_BB, _HALF, _BS)) * f3
    cosv, sinv = _sincos(phase)
    re_ref[...] = cosv
    im_ref[...] = sinv


def _dense(x0, x1, vt, consts, fmat):
    grid = (_BATCH // _BB, _SEQ // _BS)
    return pl.pallas_call(
        _dense_body,
        grid=grid,
        in_specs=[
            pl.BlockSpec((_BB, _BS), lambda i, j: (i, j)),
            pl.BlockSpec((_BB, _BS), lambda i, j: (i, j)),
            pl.BlockSpec((_BB, _BS), lambda i, j: (i, j)),
            pl.BlockSpec((8, _EMB), lambda i, j: (0, 0)),
            pl.BlockSpec((_HALF, _BS), lambda i, j: (0, j)),
        ],
        out_specs=[
            pl.BlockSpec((_BB, _BS, _EMB), lambda i, j: (i, j, 0)),
            pl.BlockSpec((_BB, _HALF, _BS), lambda i, j: (i, 0, j)),
            pl.BlockSpec((_BB, _HALF, _BS), lambda i, j: (i, 0, j)),
        ],
        out_shape=[
            jax.ShapeDtypeStruct((_BATCH, _SEQ, _EMB), jnp.float32),
            jax.ShapeDtypeStruct((_BATCH, _HALF, _SEQ), jnp.float32),
            jax.ShapeDtypeStruct((_BATCH, _HALF, _SEQ), jnp.float32),
        ],
        compiler_params=pltpu.CompilerParams(
            dimension_semantics=("parallel", "parallel"),
        ),
    )(x0, x1, vt, consts, fmat)


def kernel(x, solutions, step_info, W):
    dim = W.shape[0]
    visited_time = _make_chase()(solutions)

    x0 = x[:, :, 0]
    x1 = x[:, :, 1]

    freqs = 1.0 / (
        10000.0
        ** (jnp.arange(0, dim, 2, dtype=jnp.int32)[: dim // 2].astype(jnp.float32) / dim)
    )
    traced = (_SEQ - step_info[0] + 2 * step_info[1]).astype(jnp.float32)
    trv = jnp.broadcast_to(traced, (_EMB,))
    pad = jnp.zeros((5, _EMB), jnp.float32)
    consts = jnp.concatenate(
        [W[:, 0][None, :], W[:, 1][None, :], trv[None, :], pad], axis=0
    )
    fmat = jnp.broadcast_to(freqs[:, None], (_HALF, _SEQ))

    x_embedding, re3, im3 = _dense(x0, x1, visited_time, consts, fmat)
    freqs_cis = lax.complex(re3.transpose(0, 2, 1), im3.transpose(0, 2, 1))
    return (x_embedding, freqs_cis, visited_time)


# speedup vs baseline: 1.1759x; 1.0005x over previous
import functools
import math

import jax
import jax.numpy as jnp
from jax import lax
from jax.experimental import pallas as pl
from jax.experimental.pallas import tpu as pltpu
from jax.experimental.pallas import tpu_sc as plsc

_BATCH = 512
_SEQ = 1024
_EMB = 128
_HALF = _EMB // 2

_NC = 2
_NS = 16
_NW = _NC * _NS
_RPW = _BATCH // _NW
_LANES = 16


def _chase_body(sol_hbm, vt_hbm, sol_v, vt_v):
    wid = lax.axis_index("s") * _NC + lax.axis_index("c")
    base = wid * _RPW
    pltpu.sync_copy(sol_hbm.at[pl.ds(base, _RPW)], sol_v)

    lanes = lax.iota(jnp.int32, _LANES)
    zeros = jnp.zeros_like(lanes)

    def _zero(j, carry):
        for r in range(_RPW):
            vt_v[r, pl.ds(j * _LANES, _LANES)] = zeros
        return carry

    lax.fori_loop(0, _SEQ // _LANES, _zero, 0, unroll=False)

    def _step(i, pre):
        cur = plsc.load_gather(sol_v, [lanes, pre])
        plsc.store_scatter(vt_v, [lanes, cur], jnp.broadcast_to(i + 1, (_LANES,)))
        return cur

    lax.fori_loop(0, _SEQ + 2, _step, zeros, unroll=False)
    pltpu.sync_copy(vt_v, vt_hbm.at[pl.ds(base, _RPW)])


def _make_chase():
    mesh = plsc.VectorSubcoreMesh(
        core_axis_name="c", subcore_axis_name="s", num_cores=_NC, num_subcores=_NS
    )
    return pl.kernel(
        _chase_body,
        out_type=jax.ShapeDtypeStruct((_BATCH, _SEQ), jnp.int32),
        mesh=mesh,
        scratch_types=[
            pltpu.VMEM((_RPW, _SEQ), jnp.int32),
            pltpu.VMEM((_RPW, _SEQ), jnp.int32),
        ],
        compiler_params=pltpu.CompilerParams(
            use_tc_tiling_on_sc=False, needs_layout_passes=False
        ),
    )


_BB = 8
_BS = 512

_PIO2_HI = 1.5707855224609375
_PIO2_REST = 1.0804333896827965e-05
_2OPI = 0.6366197723675814
_S1, _S2, _S3 = -1.6666654611e-1, 8.3321608736e-3, -1.9515295891e-4
_C2, _C3, _C4 = 4.166664568298827e-2, -1.388731625493765e-3, 2.443315711809948e-5


def _sincos(phase):
    q = jnp.floor(phase * _2OPI + 0.5)
    r = (phase - q * _PIO2_HI) - q * _PIO2_REST
    r2 = r * r
    s = r + r * r2 * (_S1 + r2 * (_S2 + r2 * _S3))
    c = 1.0 + r2 * (-0.5 + r2 * (_C2 + r2 * (_C3 + r2 * _C4)))
    k = q.astype(jnp.int32) & 3
    k1 = k == 1
    k2 = k == 2
    k3 = k == 3
    cosv = jnp.where(k1, -s, jnp.where(k2, -c, jnp.where(k3, s, c)))
    sinv = jnp.where(k1, c, jnp.where(k2, -s, jnp.where(k3, -c, s)))
    return cosv, sinv


def _dense_body(x0_ref, x1_ref, vt_ref, c_ref, f_ref, emb_ref, re_ref, im_ref):
    x0 = x0_ref[...]
    x1 = x1_ref[...]
    w0 = c_ref[0, :]
    w1 = c_ref[1, :]
    emb_ref[...] = (
        x0[:, :, None] * w0[None, None, :] + x1[:, :, None] * w1[None, None, :]
    )
    tr = c_ref[2, 0]
    t2 = jnp.mod(vt_ref[...].astype(jnp.float32), tr)
    f3 = f_ref[...][None, :, :]
    phase = jnp.broadcast_to(t2[:, None, :], (_BB, _HALF, _BS)) * f3
    cosv, sinv = _sincos(phase)
    re_ref[...] = cosv
    im_ref[...] = sinv


def _dense(x0, x1, vt, consts, fmat):
    grid = (_SEQ // _BS, _BATCH // _BB)
    return pl.pallas_call(
        _dense_body,
        grid=grid,
        in_specs=[
            pl.BlockSpec((_BB, _BS), lambda j, i: (i, j)),
            pl.BlockSpec((_BB, _BS), lambda j, i: (i, j)),
            pl.BlockSpec((_BB, _BS), lambda j, i: (i, j)),
            pl.BlockSpec((8, _EMB), lambda j, i: (0, 0)),
            pl.BlockSpec((_HALF, _BS), lambda j, i: (0, j)),
        ],
        out_specs=[
            pl.BlockSpec((_BB, _BS, _EMB), lambda j, i: (i, j, 0)),
            pl.BlockSpec((_BB, _HALF, _BS), lambda j, i: (i, 0, j)),
            pl.BlockSpec((_BB, _HALF, _BS), lambda j, i: (i, 0, j)),
        ],
        out_shape=[
            jax.ShapeDtypeStruct((_BATCH, _SEQ, _EMB), jnp.float32),
            jax.ShapeDtypeStruct((_BATCH, _HALF, _SEQ), jnp.float32),
            jax.ShapeDtypeStruct((_BATCH, _HALF, _SEQ), jnp.float32),
        ],
        compiler_params=pltpu.CompilerParams(
            dimension_semantics=("parallel", "parallel"),
        ),
    )(x0, x1, vt, consts, fmat)


def kernel(x, solutions, step_info, W):
    dim = W.shape[0]
    visited_time = _make_chase()(solutions)

    x0 = x[:, :, 0]
    x1 = x[:, :, 1]

    freqs = 1.0 / (
        10000.0
        ** (jnp.arange(0, dim, 2, dtype=jnp.int32)[: dim // 2].astype(jnp.float32) / dim)
    )
    traced = (_SEQ - step_info[0] + 2 * step_info[1]).astype(jnp.float32)
    trv = jnp.broadcast_to(traced, (_EMB,))
    pad = jnp.zeros((5, _EMB), jnp.float32)
    consts = jnp.concatenate(
        [W[:, 0][None, :], W[:, 1][None, :], trv[None, :], pad], axis=0
    )
    fmat = jnp.broadcast_to(freqs[:, None], (_HALF, _SEQ))

    x_embedding, re3, im3 = _dense(x0, x1, visited_time, consts, fmat)
    freqs_cis = lax.complex(re3.transpose(0, 2, 1), im3.transpose(0, 2, 1))
    return (x_embedding, freqs_cis, visited_time)


# speedup vs baseline: 1.1876x; 1.0099x over previous
import functools
import math

import jax
import jax.numpy as jnp
from jax import lax
from jax.experimental import pallas as pl
from jax.experimental.pallas import tpu as pltpu
from jax.experimental.pallas import tpu_sc as plsc

_BATCH = 512
_SEQ = 1024
_EMB = 128
_HALF = _EMB // 2

_NC = 2
_NS = 16
_NW = _NC * _NS
_RPW = _BATCH // _NW
_LANES = 16


def _chase_body(sol_hbm, vt_hbm, sol_v, vt_v):
    wid = lax.axis_index("s") * _NC + lax.axis_index("c")
    base = wid * _RPW
    pltpu.sync_copy(sol_hbm.at[pl.ds(base, _RPW)], sol_v)

    lanes = lax.iota(jnp.int32, _LANES)
    zeros = jnp.zeros_like(lanes)

    def _zero(j, carry):
        for r in range(_RPW):
            vt_v[r, pl.ds(j * _LANES, _LANES)] = zeros
        return carry

    lax.fori_loop(0, _SEQ // _LANES, _zero, 0, unroll=False)

    def _step(i, pre):
        cur = plsc.load_gather(sol_v, [lanes, pre])
        plsc.store_scatter(vt_v, [lanes, cur], jnp.broadcast_to(i + 1, (_LANES,)))
        return cur

    lax.fori_loop(0, _SEQ + 2, _step, zeros, unroll=False)
    pltpu.sync_copy(vt_v, vt_hbm.at[pl.ds(base, _RPW)])


def _make_chase():
    mesh = plsc.VectorSubcoreMesh(
        core_axis_name="c", subcore_axis_name="s", num_cores=_NC, num_subcores=_NS
    )
    return pl.kernel(
        _chase_body,
        out_type=jax.ShapeDtypeStruct((_BATCH, _SEQ), jnp.int32),
        mesh=mesh,
        scratch_types=[
            pltpu.VMEM((_RPW, _SEQ), jnp.int32),
            pltpu.VMEM((_RPW, _SEQ), jnp.int32),
        ],
        compiler_params=pltpu.CompilerParams(
            use_tc_tiling_on_sc=False, needs_layout_passes=False
        ),
    )


_BB = 8
_BS = 256

_PIO2_HI = 1.5707855224609375
_PIO2_REST = 1.0804333896827965e-05
_2OPI = 0.6366197723675814
_S1, _S2, _S3 = -1.6666654611e-1, 8.3321608736e-3, -1.9515295891e-4
_C2, _C3, _C4 = 4.166664568298827e-2, -1.388731625493765e-3, 2.443315711809948e-5


def _sincos(phase):
    q = jnp.floor(phase * _2OPI + 0.5)
    r = (phase - q * _PIO2_HI) - q * _PIO2_REST
    r2 = r * r
    s = r + r * r2 * (_S1 + r2 * (_S2 + r2 * _S3))
    c = 1.0 + r2 * (-0.5 + r2 * (_C2 + r2 * (_C3 + r2 * _C4)))
    k = q.astype(jnp.int32) & 3
    k1 = k == 1
    k2 = k == 2
    k3 = k == 3
    cosv = jnp.where(k1, -s, jnp.where(k2, -c, jnp.where(k3, s, c)))
    sinv = jnp.where(k1, c, jnp.where(k2, -s, jnp.where(k3, -c, s)))
    return cosv, sinv


def _dense_body(x0_ref, x1_ref, vt_ref, c_ref, f_ref, emb_ref, re_ref, im_ref):
    x0 = x0_ref[...]
    x1 = x1_ref[...]
    w0 = c_ref[0, :]
    w1 = c_ref[1, :]
    emb_ref[...] = (
        x0[:, :, None] * w0[None, None, :] + x1[:, :, None] * w1[None, None, :]
    )
    tr = c_ref[2, 0]
    t2 = jnp.mod(vt_ref[...].astype(jnp.float32), tr)
    f3 = f_ref[...][None, :, :]
    phase = jnp.broadcast_to(t2[:, None, :], (_BB, _HALF, _BS)) * f3
    cosv, sinv = _sincos(phase)
    re_ref[...] = cosv
    im_ref[...] = sinv


def _dense(x0, x1, vt, consts, fmat):
    grid = (_SEQ // _BS, _BATCH // _BB)
    return pl.pallas_call(
        _dense_body,
        grid=grid,
        in_specs=[
            pl.BlockSpec((_BB, _BS), lambda j, i: (i, j)),
            pl.BlockSpec((_BB, _BS), lambda j, i: (i, j)),
            pl.BlockSpec((_BB, _BS), lambda j, i: (i, j)),
            pl.BlockSpec((8, _EMB), lambda j, i: (0, 0)),
            pl.BlockSpec((_HALF, _BS), lambda j, i: (0, j)),
        ],
        out_specs=[
            pl.BlockSpec((_BB, _BS, _EMB), lambda j, i: (i, j, 0)),
            pl.BlockSpec((_BB, _HALF, _BS), lambda j, i: (i, 0, j)),
            pl.BlockSpec((_BB, _HALF, _BS), lambda j, i: (i, 0, j)),
        ],
        out_shape=[
            jax.ShapeDtypeStruct((_BATCH, _SEQ, _EMB), jnp.float32),
            jax.ShapeDtypeStruct((_BATCH, _HALF, _SEQ), jnp.float32),
            jax.ShapeDtypeStruct((_BATCH, _HALF, _SEQ), jnp.float32),
        ],
        compiler_params=pltpu.CompilerParams(
            dimension_semantics=("parallel", "parallel"),
        ),
    )(x0, x1, vt, consts, fmat)


def kernel(x, solutions, step_info, W):
    dim = W.shape[0]
    visited_time = _make_chase()(solutions)

    x0 = x[:, :, 0]
    x1 = x[:, :, 1]

    freqs = 1.0 / (
        10000.0
        ** (jnp.arange(0, dim, 2, dtype=jnp.int32)[: dim // 2].astype(jnp.float32) / dim)
    )
    traced = (_SEQ - step_info[0] + 2 * step_info[1]).astype(jnp.float32)
    trv = jnp.broadcast_to(traced, (_EMB,))
    pad = jnp.zeros((5, _EMB), jnp.float32)
    consts = jnp.concatenate(
        [W[:, 0][None, :], W[:, 1][None, :], trv[None, :], pad], axis=0
    )
    fmat = jnp.broadcast_to(freqs[:, None], (_HALF, _SEQ))

    x_embedding, re3, im3 = _dense(x0, x1, visited_time, consts, fmat)
    freqs_cis = lax.complex(re3.transpose(0, 2, 1), im3.transpose(0, 2, 1))
    return (x_embedding, freqs_cis, visited_time)


# speedup vs baseline: 1.1967x; 1.0077x over previous
import math

import jax
import jax.numpy as jnp
from jax import lax
from jax.experimental import pallas as pl
from jax.experimental.pallas import tpu as pltpu
from jax.experimental.pallas import tpu_sc as plsc

_BATCH = 512
_SEQ = 1024
_EMB = 128
_HALF = _EMB // 2

_NC = 2
_NS = 16
_NW = _NC * _NS
_RPW = _BATCH // _NW
_LANES = 16


def _chase_body(sol_hbm, vt_hbm, sol_v, vt_v):
    wid = lax.axis_index("s") * _NC + lax.axis_index("c")
    base = wid * _RPW
    pltpu.sync_copy(sol_hbm.at[pl.ds(base, _RPW)], sol_v)

    lanes = lax.iota(jnp.int32, _LANES)
    zeros = jnp.zeros_like(lanes)

    def _zero(j, carry):
        for r in range(_RPW):
            vt_v[r, pl.ds(j * _LANES, _LANES)] = zeros
        return carry

    lax.fori_loop(0, _SEQ // _LANES, _zero, 0, unroll=False)

    def _step(i, pre):
        cur = plsc.load_gather(sol_v, [lanes, pre])
        plsc.store_scatter(vt_v, [lanes, cur], jnp.broadcast_to(i + 1, (_LANES,)))
        return cur

    lax.fori_loop(0, _SEQ + 2, _step, zeros, unroll=False)
    pltpu.sync_copy(vt_v, vt_hbm.at[pl.ds(base, _RPW)])


def _make_chase():
    mesh = plsc.VectorSubcoreMesh(
        core_axis_name="c", subcore_axis_name="s", num_cores=_NC, num_subcores=_NS
    )
    return pl.kernel(
        _chase_body,
        out_type=jax.ShapeDtypeStruct((_BATCH, _SEQ), jnp.int32),
        mesh=mesh,
        scratch_types=[
            pltpu.VMEM((_RPW, _SEQ), jnp.int32),
            pltpu.VMEM((_RPW, _SEQ), jnp.int32),
        ],
        compiler_params=pltpu.CompilerParams(
            use_tc_tiling_on_sc=False, needs_layout_passes=False
        ),
    )


_BB = 8
_BS = 128

_PIO2_HI = 1.5707855224609375
_PIO2_REST = 1.0804333896827965e-05
_2OPI = 0.6366197723675814
_S1, _S2, _S3 = -1.6666654611e-1, 8.3321608736e-3, -1.9515295891e-4
_C2, _C3, _C4 = 4.166664568298827e-2, -1.388731625493765e-3, 2.443315711809948e-5


def _sincos(phase):
    q = jnp.floor(phase * _2OPI + 0.5)
    r = (phase - q * _PIO2_HI) - q * _PIO2_REST
    r2 = r * r
    s = r + r * r2 * (_S1 + r2 * (_S2 + r2 * _S3))
    c = 1.0 + r2 * (-0.5 + r2 * (_C2 + r2 * (_C3 + r2 * _C4)))
    k = q.astype(jnp.int32) & 3
    k1 = k == 1
    k2 = k == 2
    k3 = k == 3
    cosv = jnp.where(k1, -s, jnp.where(k2, -c, jnp.where(k3, s, c)))
    sinv = jnp.where(k1, c, jnp.where(k2, -s, jnp.where(k3, -c, s)))
    return cosv, sinv


def _dense_body(x0_ref, x1_ref, vt_ref, c_ref, f_ref, emb_ref, re_ref, im_ref):
    x0 = x0_ref[...]
    x1 = x1_ref[...]
    w0 = c_ref[0, :]
    w1 = c_ref[1, :]
    emb_ref[...] = (
        x0[:, :, None] * w0[None, None, :] + x1[:, :, None] * w1[None, None, :]
    )
    tr = c_ref[2, 0]
    t2 = jnp.mod(vt_ref[...].astype(jnp.float32), tr)
    f3 = f_ref[...][None, :, :]
    phase = jnp.broadcast_to(t2[:, None, :], (_BB, _HALF, _BS)) * f3
    cosv, sinv = _sincos(phase)
    re_ref[...] = cosv
    im_ref[...] = sinv


def _dense(x0, x1, vt, consts, fmat):
    grid = (_SEQ // _BS, _BATCH // _BB)
    return pl.pallas_call(
        _dense_body,
        grid=grid,
        in_specs=[
            pl.BlockSpec((_BB, _BS), lambda j, i: (i, j)),
            pl.BlockSpec((_BB, _BS), lambda j, i: (i, j)),
            pl.BlockSpec((_BB, _BS), lambda j, i: (i, j)),
            pl.BlockSpec((8, _EMB), lambda j, i: (0, 0)),
            pl.BlockSpec((_HALF, _BS), lambda j, i: (0, j)),
        ],
        out_specs=[
            pl.BlockSpec((_BB, _BS, _EMB), lambda j, i: (i, j, 0)),
            pl.BlockSpec((_BB, _HALF, _BS), lambda j, i: (i, 0, j)),
            pl.BlockSpec((_BB, _HALF, _BS), lambda j, i: (i, 0, j)),
        ],
        out_shape=[
            jax.ShapeDtypeStruct((_BATCH, _SEQ, _EMB), jnp.float32),
            jax.ShapeDtypeStruct((_BATCH, _HALF, _SEQ), jnp.float32),
            jax.ShapeDtypeStruct((_BATCH, _HALF, _SEQ), jnp.float32),
        ],
        compiler_params=pltpu.CompilerParams(
            dimension_semantics=("parallel", "parallel"),
        ),
    )(x0, x1, vt, consts, fmat)


def kernel(x, solutions, step_info, W):
    dim = W.shape[0]
    visited_time = _make_chase()(solutions)

    x0 = x[:, :, 0]
    x1 = x[:, :, 1]

    freqs = 1.0 / (
        10000.0
        ** (jnp.arange(0, dim, 2, dtype=jnp.int32)[: dim // 2].astype(jnp.float32) / dim)
    )
    traced = (_SEQ - step_info[0] + 2 * step_info[1]).astype(jnp.float32)
    trv = jnp.broadcast_to(traced, (_EMB,))
    pad = jnp.zeros((5, _EMB), jnp.float32)
    consts = jnp.concatenate(
        [W[:, 0][None, :], W[:, 1][None, :], trv[None, :], pad], axis=0
    )
    fmat = jnp.broadcast_to(freqs[:, None], (_HALF, _SEQ))

    x_embedding, re3, im3 = _dense(x0, x1, visited_time, consts, fmat)
    freqs_cis = lax.complex(re3.transpose(0, 2, 1), im3.transpose(0, 2, 1))
    return (x_embedding, freqs_cis, visited_time)
